# Initial kernel scaffold; baseline (speedup 1.0000x reference)
import jax
import jax.numpy as jnp
from jax.experimental import pallas as pl


def kernel(edge_weight, src, dst):
    raise NotImplementedError("write your pallas kernel here")


# speedup vs baseline: 210.3540x; 210.3540x over previous
import functools

import jax
import jax.numpy as jnp
from jax import lax
from jax.experimental import pallas as pl
from jax.experimental.pallas import tpu as pltpu
from jax.experimental.pallas import tpu_sc as plsc

_N = 50000
_E = 1600000
_NUM_EIG = 4
_K = 30
_TOL = 1e-10

_NW = 32
_EP = _E // _NW
_CH = 2000
_NCH = _EP // _CH
_NPAD = 50176


def _sc_matvec_body(src_hbm, dst_hbm, wt_hbm, x_hbm, part_hbm,
                    x_vm, wpart_vm, sbuf, dbuf, wbuf):
    wid = lax.axis_index("s") * 2 + lax.axis_index("c")
    pltpu.sync_copy(x_hbm, x_vm)

    def zero_body(j, c):
        wpart_vm[pl.ds(j * 16, 16)] = jnp.zeros((16,), jnp.float32)
        return c

    lax.fori_loop(0, _NPAD // 16, zero_body, 0)

    def chunk_body(c, carry):
        base = wid * _EP + c * _CH
        pltpu.sync_copy(src_hbm.at[pl.ds(base, _CH)], sbuf)
        pltpu.sync_copy(dst_hbm.at[pl.ds(base, _CH)], dbuf)
        pltpu.sync_copy(wt_hbm.at[pl.ds(base, _CH)], wbuf)

        def group_body(g, cc):
            o = g * 16
            s16 = sbuf[pl.ds(o, 16)]
            d16 = dbuf[pl.ds(o, 16)]
            w16 = wbuf[pl.ds(o, 16)]
            xs = plsc.load_gather(x_vm, [d16])
            plsc.addupdate_scatter(wpart_vm, [s16], w16 * xs)
            xd = plsc.load_gather(x_vm, [s16])
            plsc.addupdate_scatter(wpart_vm, [d16], w16 * xd)
            return cc

        lax.fori_loop(0, _CH // 16, group_body, 0)
        return carry

    lax.fori_loop(0, _NCH, chunk_body, 0)
    pltpu.sync_copy(wpart_vm, part_hbm.at[wid])


@functools.partial(jax.jit, static_argnums=())
def _sc_matvec(src, dst, wt, x_pad):
    mesh = plsc.VectorSubcoreMesh(core_axis_name="c", subcore_axis_name="s")
    f = pl.kernel(
        _sc_matvec_body,
        out_type=jax.ShapeDtypeStruct((_NW, _NPAD), jnp.float32),
        mesh=mesh,
        compiler_params=pltpu.CompilerParams(needs_layout_passes=False),
        scratch_types=[
            pltpu.VMEM((_NPAD,), jnp.float32),
            pltpu.VMEM((_NPAD,), jnp.float32),
            pltpu.VMEM((_CH,), jnp.int32),
            pltpu.VMEM((_CH,), jnp.int32),
            pltpu.VMEM((_CH,), jnp.float32),
        ],
    )
    return f(src, dst, wt, x_pad)


def kernel(edge_weight, src, dst):
    n = _N
    k = min(_K, n - 1)
    src = src.astype(jnp.int32)
    dst = dst.astype(jnp.int32)
    wt = edge_weight.astype(jnp.float32)

    def adj_matvec(x):
        x_pad = jnp.zeros(_NPAD, jnp.float32).at[:n].set(x)
        parts = _sc_matvec(src, dst, wt, x_pad)
        return parts.sum(axis=0)[:n]

    deg = adj_matvec(jnp.ones(n, jnp.float32))

    def matvec(x):
        return deg * x - adj_matvec(x)

    v = jax.random.normal(jax.random.key(42), (n,), dtype=jnp.float32)
    v = v - v.mean()
    v_norm = jnp.linalg.norm(v)
    v_alt = jnp.ones(n, jnp.float32).at[::2].set(-1.0)
    v_alt = v_alt - v_alt.mean()
    v_alt_norm = jnp.linalg.norm(v_alt)
    v = jnp.where(v_norm < _TOL, v_alt / v_alt_norm, v / v_norm)
    V = jnp.zeros((n, k + 1), jnp.float32).at[:, 0].set(v)
    alphas = jnp.zeros(k, jnp.float32)
    betas = jnp.zeros(k, jnp.float32)

    def body(i, carry):
        V, alphas, betas, done, actual_k = carry
        w = matvec(V[:, i])
        alpha = jnp.dot(V[:, i], w)
        alphas_n = alphas.at[i].set(alpha)
        w = w - alpha * V[:, i]
        im1 = jnp.maximum(i - 1, 0)
        w = jnp.where(i > 0, w - betas[im1] * V[:, im1], w)
        coeffs = V.T @ w
        w = jnp.where(i > 0, w - V @ coeffs, w)
        w = w - w.mean()
        beta = jnp.linalg.norm(w)
        small = beta < _TOL
        active = jnp.logical_not(done)
        step_ok = active & jnp.logical_not(small)
        alphas = jnp.where(active, alphas_n, alphas)
        betas = jnp.where(step_ok, betas.at[i].set(beta), betas)
        V = jnp.where(step_ok, V.at[:, i + 1].set(w / beta), V)
        actual_k = jnp.where(active & small, i + 1, actual_k)
        done = jnp.logical_or(done, small)
        return V, alphas, betas, done, actual_k

    V, alphas, betas, done, actual_k = jax.lax.fori_loop(
        0, k, body,
        (V, alphas, betas, jnp.array(False), jnp.array(k, jnp.int32)),
    )
    T = jnp.diag(alphas)
    off = jnp.arange(k - 1)
    T = T.at[off, off + 1].set(betas[:k - 1])
    T = T.at[off + 1, off].set(betas[:k - 1])
    eigenvalues, eigenvectors_T = jnp.linalg.eigh(T)
    mask = eigenvalues > 1e-06
    count = jnp.sum(mask.astype(jnp.int32))
    take = jnp.minimum(_NUM_EIG, count)
    pos = jnp.where(mask, jnp.cumsum(mask.astype(jnp.int32)) - 1, _NUM_EIG)
    pos = jnp.minimum(pos, _NUM_EIG)
    buf = jnp.zeros(_NUM_EIG + 1, jnp.int32).at[pos].set(
        jnp.arange(k, dtype=jnp.int32))
    sel = buf[:_NUM_EIG]
    slot_ok = jnp.arange(_NUM_EIG) < take
    sel_vals = jnp.where(slot_ok, eigenvalues[sel], 0.0)
    Y = jnp.where(slot_ok[None, :], eigenvectors_T[:, sel], 0.0)
    evecs = V[:, :k] @ Y
    ok = actual_k >= 2
    evecs = jnp.where(ok, evecs, jnp.zeros((n, _NUM_EIG), jnp.float32))
    sel_vals = jnp.where(ok, sel_vals, jnp.zeros(_NUM_EIG, jnp.float32))
    return (evecs, sel_vals)


# speedup vs baseline: 468.9339x; 2.2293x over previous
import functools

import jax
import jax.numpy as jnp
from jax import lax
from jax.experimental import pallas as pl
from jax.experimental.pallas import tpu as pltpu
from jax.experimental.pallas import tpu_sc as plsc

_N = 50000
_E = 1600000
_NUM_EIG = 4
_K = 30
_TOL = 1e-10

_NW = 32
_EP = _E // _NW
_CH = 2000
_NCH = _EP // _CH
_NG = _CH // 16
_NPAD = 50176
_NROW = _NPAD // 128
_RCH = 98
_NRCH = _NROW // _RCH


def _sc_matvec_body(edges_hbm, x_hbm, ridx_hbm, part_hbm,
                    x_vm, wpart_vm, eb0, eb1, ridx_vm, wsum_sh,
                    semx, sem0, sem1):
    cid = lax.axis_index("c")
    sid = lax.axis_index("s")
    wid = sid * 2 + cid

    descx = pltpu.async_copy(x_hbm, x_vm, semx)
    pltpu.sync_copy(ridx_hbm, ridx_vm)

    @plsc.parallel_loop(0, _NROW * 8, unroll=8)
    def _zero(j):
        wpart_vm[j // 8, pl.ds((j % 8) * 16, 16)] = jnp.zeros(
            (16,), jnp.float32)

    @pl.when(sid == 0)
    def _():
        pltpu.sync_copy(wpart_vm, wsum_sh)

    plsc.subcore_barrier()
    descx.wait()

    descs = [None, None]
    bufs = (eb0, eb1)
    sems = (sem0, sem1)
    descs[0] = pltpu.async_copy(edges_hbm.at[wid, 0], eb0, sem0)
    for c in range(_NCH):
        b = c % 2
        if c + 1 < _NCH:
            nb = (c + 1) % 2
            descs[nb] = pltpu.async_copy(
                edges_hbm.at[wid, c + 1], bufs[nb], sems[nb])
        descs[b].wait()
        eb = bufs[b]

        @plsc.parallel_loop(0, _NG, unroll=5)
        def _groups(g):
            o = g * 16
            s16 = eb[0, pl.ds(o, 16)]
            d16 = eb[1, pl.ds(o, 16)]
            w16 = plsc.bitcast(eb[2, pl.ds(o, 16)], jnp.float32)
            sr = lax.shift_right_logical(s16, 7)
            sc_ = lax.bitwise_and(s16, 127)
            dr = lax.shift_right_logical(d16, 7)
            dc = lax.bitwise_and(d16, 127)
            xs = plsc.load_gather(x_vm, [d16])
            plsc.addupdate_scatter(wpart_vm, [sr, sc_], w16 * xs)
            xd = plsc.load_gather(x_vm, [s16])
            plsc.addupdate_scatter(wpart_vm, [dr, dc], w16 * xd)

    for j in range(_NRCH):
        pltpu.sync_copy(wpart_vm.at[pl.ds(j * _RCH, _RCH)],
                        wsum_sh.at[ridx_vm.at[j]], add=True)
    plsc.subcore_barrier()

    @pl.when(sid == 0)
    def _():
        pltpu.sync_copy(wsum_sh, part_hbm.at[cid])


def _sc_matvec(edges3, x_pad, ridx):
    mesh = plsc.VectorSubcoreMesh(core_axis_name="c", subcore_axis_name="s")
    f = pl.kernel(
        _sc_matvec_body,
        out_type=jax.ShapeDtypeStruct((2, _NROW, 128), jnp.float32),
        mesh=mesh,
        compiler_params=pltpu.CompilerParams(needs_layout_passes=False),
        scratch_types=[
            pltpu.VMEM((_NPAD,), jnp.float32),
            pltpu.VMEM((_NROW, 128), jnp.float32),
            pltpu.VMEM((3, _CH), jnp.int32),
            pltpu.VMEM((3, _CH), jnp.int32),
            pltpu.VMEM((_NRCH, _RCH), jnp.int32),
            pltpu.VMEM_SHARED((_NROW, 128), jnp.float32),
            pltpu.SemaphoreType.DMA,
            pltpu.SemaphoreType.DMA,
            pltpu.SemaphoreType.DMA,
        ],
    )
    return f(edges3, x_pad, ridx)


def kernel(edge_weight, src, dst):
    n = _N
    k = min(_K, n - 1)
    src = src.astype(jnp.int32)
    dst = dst.astype(jnp.int32)
    wt = edge_weight.astype(jnp.float32)
    wt_bits = lax.bitcast_convert_type(wt, jnp.int32)
    edges3 = jnp.stack(
        [src.reshape(_NW, _NCH, _CH),
         dst.reshape(_NW, _NCH, _CH),
         wt_bits.reshape(_NW, _NCH, _CH)], axis=2)
    ridx = (jnp.arange(_NROW, dtype=jnp.int32)).reshape(_NRCH, _RCH)

    def adj_matvec(x):
        x_pad = jnp.zeros(_NPAD, jnp.float32).at[:n].set(x)
        parts = _sc_matvec(edges3, x_pad, ridx)
        return parts.sum(axis=0).reshape(_NPAD)[:n]

    deg = adj_matvec(jnp.ones(n, jnp.float32))

    def matvec(x):
        return deg * x - adj_matvec(x)

    v = jax.random.normal(jax.random.key(42), (n,), dtype=jnp.float32)
    v = v - v.mean()
    v_norm = jnp.linalg.norm(v)
    v_alt = jnp.ones(n, jnp.float32).at[::2].set(-1.0)
    v_alt = v_alt - v_alt.mean()
    v_alt_norm = jnp.linalg.norm(v_alt)
    v = jnp.where(v_norm < _TOL, v_alt / v_alt_norm, v / v_norm)
    V = jnp.zeros((n, k + 1), jnp.float32).at[:, 0].set(v)
    alphas = jnp.zeros(k, jnp.float32)
    betas = jnp.zeros(k, jnp.float32)

    def body(i, carry):
        V, alphas, betas, done, actual_k = carry
        w = matvec(V[:, i])
        alpha = jnp.dot(V[:, i], w)
        alphas_n = alphas.at[i].set(alpha)
        w = w - alpha * V[:, i]
        im1 = jnp.maximum(i - 1, 0)
        w = jnp.where(i > 0, w - betas[im1] * V[:, im1], w)
        coeffs = V.T @ w
        w = jnp.where(i > 0, w - V @ coeffs, w)
        w = w - w.mean()
        beta = jnp.linalg.norm(w)
        small = beta < _TOL
        active = jnp.logical_not(done)
        step_ok = active & jnp.logical_not(small)
        alphas = jnp.where(active, alphas_n, alphas)
        betas = jnp.where(step_ok, betas.at[i].set(beta), betas)
        V = jnp.where(step_ok, V.at[:, i + 1].set(w / beta), V)
        actual_k = jnp.where(active & small, i + 1, actual_k)
        done = jnp.logical_or(done, small)
        return V, alphas, betas, done, actual_k

    V, alphas, betas, done, actual_k = jax.lax.fori_loop(
        0, k, body,
        (V, alphas, betas, jnp.array(False), jnp.array(k, jnp.int32)),
    )
    T = jnp.diag(alphas)
    off = jnp.arange(k - 1)
    T = T.at[off, off + 1].set(betas[:k - 1])
    T = T.at[off + 1, off].set(betas[:k - 1])
    eigenvalues, eigenvectors_T = jnp.linalg.eigh(T)
    mask = eigenvalues > 1e-06
    count = jnp.sum(mask.astype(jnp.int32))
    take = jnp.minimum(_NUM_EIG, count)
    pos = jnp.where(mask, jnp.cumsum(mask.astype(jnp.int32)) - 1, _NUM_EIG)
    pos = jnp.minimum(pos, _NUM_EIG)
    buf = jnp.zeros(_NUM_EIG + 1, jnp.int32).at[pos].set(
        jnp.arange(k, dtype=jnp.int32))
    sel = buf[:_NUM_EIG]
    slot_ok = jnp.arange(_NUM_EIG) < take
    sel_vals = jnp.where(slot_ok, eigenvalues[sel], 0.0)
    Y = jnp.where(slot_ok[None, :], eigenvectors_T[:, sel], 0.0)
    evecs = V[:, :k] @ Y
    ok = actual_k >= 2
    evecs = jnp.where(ok, evecs, jnp.zeros((n, _NUM_EIG), jnp.float32))
    sel_vals = jnp.where(ok, sel_vals, jnp.zeros(_NUM_EIG, jnp.float32))
    return (evecs, sel_vals)
